# Initial kernel scaffold; baseline (speedup 1.0000x reference)
import jax
import jax.numpy as jnp
from jax.experimental import pallas as pl


def kernel(x, edge_index, proj, gin_params):
    raise NotImplementedError("write your pallas kernel here")


# speedup vs baseline: 4.9287x; 4.9287x over previous
import functools

import jax
import jax.numpy as jnp
from jax import lax
from jax.experimental import pallas as pl
from jax.experimental.pallas import tpu as pltpu
from jax.experimental.pallas import tpu_sc as plsc

N = 10000
E = 320000
DH = 128
NC = 2
NS = 16
EB = 80
CB = 25

NBLK1 = E // (NS * EB)
NCH1 = NBLK1 // CB
NBLK0 = E // (NC * NS * EB)
NCH0 = NBLK0 // CB

RPS = 624
RTAIL = N - NS * RPS

BR = 2000
GR = N // BR

_INV_N = 1.0 / N


def _store_sums(i, z, s1_ref, s2_ref):
    ps = jnp.sum(z, axis=0, keepdims=True)
    psq = jnp.sum(z * z, axis=0, keepdims=True)

    @pl.when(i == 0)
    def _():
        s1_ref[...] = ps
        s2_ref[...] = psq

    @pl.when(i != 0)
    def _():
        s1_ref[...] = s1_ref[...] + ps
        s2_ref[...] = s2_ref[...] + psq


def _norm_gelu(z, s1_ref, s2_ref, g_ref, be_ref, al_ref):
    al = al_ref[...]
    m = s1_ref[...] * _INV_N
    e2 = s2_ref[...] * _INV_N
    am = al * m
    var = e2 - 2.0 * am * m + am * am
    zn = (z - am) * jax.lax.rsqrt(var + 1e-5) * g_ref[...] + be_ref[...]
    return jax.nn.gelu(zn)


def _sums_shapes(d):
    return (jax.ShapeDtypeStruct((1, d), jnp.float32),
            jax.ShapeDtypeStruct((1, d), jnp.float32))


def _sums_specs(d):
    return (pl.BlockSpec((1, d), lambda i: (0, 0)),
            pl.BlockSpec((1, d), lambda i: (0, 0)))


def _row_spec(d):
    return pl.BlockSpec((BR, d), lambda i: (i, 0))


def _fs(a):
    return pl.BlockSpec(a.shape, lambda i: (0,) * a.ndim)


def _proj_body(x_ref, wp_ref, bp_ref, o_ref):
    h = jnp.dot(x_ref[...], wp_ref[...], preferred_element_type=jnp.float32)
    o_ref[...] = h + bp_ref[...]


def _proj(x, Wp, bp):
    bp2 = bp.reshape(1, -1)
    return pl.pallas_call(
        _proj_body,
        grid=(GR,),
        in_specs=[_row_spec(Wp.shape[0]), _fs(Wp), _fs(bp2)],
        out_specs=_row_spec(Wp.shape[1]),
        out_shape=jax.ShapeDtypeStruct((N, Wp.shape[1]), jnp.float32),
    )(x, Wp, bp2)


def _first0_body(h_ref, s_ref, w_ref, b_ref, o_ref, s1_ref, s2_ref):
    hin = h_ref[...] + (s_ref[0] + s_ref[1])
    z = jnp.dot(hin, w_ref[...], preferred_element_type=jnp.float32)
    z = z + b_ref[...]
    o_ref[...] = z
    _store_sums(pl.program_id(0), z, s1_ref, s2_ref)


def _linear_first0(h0, s, W, b):
    dout = W.shape[1]
    b2 = b.reshape(1, dout)
    return pl.pallas_call(
        _first0_body,
        grid=(GR,),
        in_specs=[_row_spec(DH),
                  pl.BlockSpec((2, BR, DH), lambda i: (0, i, 0)),
                  _fs(W), _fs(b2)],
        out_specs=(_row_spec(dout),) + _sums_specs(dout),
        out_shape=(jax.ShapeDtypeStruct((N, dout), jnp.float32),)
        + _sums_shapes(dout),
    )(h0, s, W, b2)


def _first1_body(s_ref, w_ref, b_ref, o_ref, s1_ref, s2_ref):
    h = jnp.concatenate([s_ref[0], s_ref[1]], axis=1)
    z = jnp.dot(h, w_ref[...], preferred_element_type=jnp.float32)
    z = z + b_ref[...]
    o_ref[...] = z
    _store_sums(pl.program_id(0), z, s1_ref, s2_ref)


def _linear_first1(s, W, b):
    dout = W.shape[1]
    b2 = b.reshape(1, dout)
    return pl.pallas_call(
        _first1_body,
        grid=(GR,),
        in_specs=[pl.BlockSpec((2, BR, DH), lambda i: (0, i, 0)),
                  _fs(W), _fs(b2)],
        out_specs=(_row_spec(dout),) + _sums_specs(dout),
        out_shape=(jax.ShapeDtypeStruct((N, dout), jnp.float32),)
        + _sums_shapes(dout),
    )(s, W, b2)


def _linear_body(s_ref, z1_ref, z2_ref, g_ref, be_ref, al_ref, w_ref, b_ref,
                 o_ref, s1_ref, s2_ref):
    a = _norm_gelu(s_ref[...], z1_ref, z2_ref, g_ref, be_ref, al_ref)
    zn = jnp.dot(a, w_ref[...], preferred_element_type=jnp.float32)
    zn = zn + b_ref[...]
    o_ref[...] = zn
    _store_sums(pl.program_id(0), zn, s1_ref, s2_ref)


def _linear(z, sums, norm_params, W, b):
    g, be, al = norm_params
    din, dout = W.shape
    onerow = pl.BlockSpec((1, din), lambda i: (0, 0))
    b2 = b.reshape(1, dout)
    return pl.pallas_call(
        _linear_body,
        grid=(GR,),
        in_specs=[_row_spec(din), onerow, onerow, onerow, onerow, onerow,
                  _fs(W), _fs(b2)],
        out_specs=(_row_spec(dout),) + _sums_specs(dout),
        out_shape=(jax.ShapeDtypeStruct((N, dout), jnp.float32),)
        + _sums_shapes(dout),
    )(z, sums[0], sums[1], g.reshape(1, din), be.reshape(1, din),
      al.reshape(1, din), W, b2)


def _apply_body(z_ref, z1_ref, z2_ref, g_ref, be_ref, al_ref, *out_refs):
    a = _norm_gelu(z_ref[...], z1_ref, z2_ref, g_ref, be_ref, al_ref)
    if len(out_refs) == 2:
        d = a.shape[1] // 2
        out_refs[0][...] = a[:, :d]
        out_refs[1][...] = a[:, d:]
    else:
        out_refs[0][...] = a


def _apply(z, sums, norm_params, split_out):
    g, be, al = norm_params
    d = z.shape[1]
    if split_out:
        out_specs = (_row_spec(d // 2), _row_spec(d // 2))
        out_shape = (jax.ShapeDtypeStruct((N, d // 2), jnp.float32),
                     jax.ShapeDtypeStruct((N, d // 2), jnp.float32))
    else:
        out_specs = _row_spec(d)
        out_shape = jax.ShapeDtypeStruct((N, d), jnp.float32)
    onerow = pl.BlockSpec((1, d), lambda i: (0, 0))
    return pl.pallas_call(
        _apply_body,
        grid=(GR,),
        in_specs=[_row_spec(d), onerow, onerow, onerow, onerow, onerow],
        out_specs=out_specs,
        out_shape=out_shape,
    )(z, sums[0], sums[1], g.reshape(1, d), be.reshape(1, d),
      al.reshape(1, d))


def _acc_init(table, acc_sh, sid, r0):
    pltpu.sync_copy(table.at[pl.ds(r0, RPS)], acc_sh.at[pl.ds(r0, RPS)])

    @pl.when(sid == NS - 1)
    def _():
        pltpu.sync_copy(table.at[pl.ds(NS * RPS, RTAIL)],
                        acc_sh.at[pl.ds(NS * RPS, RTAIL)])


def _acc_flush(acc_sh, out_hbm, cid, sid, r0):
    pltpu.sync_copy(acc_sh.at[pl.ds(r0, RPS)], out_hbm.at[cid, pl.ds(r0, RPS)])

    @pl.when(sid == NS - 1)
    def _():
        pltpu.sync_copy(acc_sh.at[pl.ds(NS * RPS, RTAIL)],
                        out_hbm.at[cid, pl.ds(NS * RPS, RTAIL)])


def _edge_chunks(table, ei_hbm, widx, nchunks, acc_sh, src_v, dst_v, rows_v,
                 sem):
    @pl.loop(0, nchunks)
    def _(c):
        pltpu.sync_copy(ei_hbm.at[0, widx, c], src_v)
        pltpu.sync_copy(ei_hbm.at[1, widx, c], dst_v)

        @pl.loop(0, CB)
        def _(j):
            pltpu.async_copy(table.at[src_v.at[j]], rows_v, sem).wait()
            pltpu.sync_copy(rows_v, acc_sh.at[dst_v.at[j]], add=True)


@functools.cache
def _make_segsum0():
    mesh = plsc.VectorSubcoreMesh(core_axis_name="c", subcore_axis_name="s")
    return jax.jit(functools.partial(
        pl.kernel,
        out_type=jax.ShapeDtypeStruct((NC, N, DH), jnp.float32),
        mesh=mesh,
        scratch_types=[
            pltpu.VMEM((CB, EB), jnp.int32),
            pltpu.VMEM((CB, EB), jnp.int32),
            pltpu.VMEM((EB, DH), jnp.float32),
            pltpu.VMEM_SHARED((N, DH), jnp.float32),
            pltpu.SemaphoreType.DMA,
        ],
    )(_segsum0_body))


def _segsum0_body(h_hbm, zeros_hbm, ei_hbm, out_hbm, src_v, dst_v, rows_v,
                  acc_sh, sem):
    cid = lax.axis_index("c")
    sid = lax.axis_index("s")
    w = cid * NS + sid
    r0 = sid * RPS
    _acc_init(zeros_hbm, acc_sh, sid, r0)
    plsc.subcore_barrier()
    _edge_chunks(h_hbm, ei_hbm, w, NCH0, acc_sh, src_v, dst_v, rows_v, sem)
    plsc.subcore_barrier()
    _acc_flush(acc_sh, out_hbm, cid, sid, r0)


@functools.cache
def _make_segsum1():
    mesh = plsc.VectorSubcoreMesh(core_axis_name="c", subcore_axis_name="s")
    return jax.jit(functools.partial(
        pl.kernel,
        out_type=jax.ShapeDtypeStruct((NC, N, DH), jnp.float32),
        mesh=mesh,
        scratch_types=[
            pltpu.VMEM((CB, EB), jnp.int32),
            pltpu.VMEM((CB, EB), jnp.int32),
            pltpu.VMEM((EB, DH), jnp.float32),
            pltpu.VMEM_SHARED((N, DH), jnp.float32),
            pltpu.SemaphoreType.DMA,
        ],
    )(_segsum1_body))


def _segsum1_body(ha_hbm, hb_hbm, ei_hbm, out_hbm, src_v, dst_v, rows_v,
                  acc_sh, sem):
    cid = lax.axis_index("c")
    sid = lax.axis_index("s")
    r0 = sid * RPS

    def run(table):
        _acc_init(table, acc_sh, sid, r0)
        plsc.subcore_barrier()
        _edge_chunks(table, ei_hbm, sid, NCH1, acc_sh, src_v, dst_v, rows_v,
                     sem)

    @pl.when(cid == 0)
    def _():
        run(ha_hbm)

    @pl.when(cid == 1)
    def _():
        run(hb_hbm)

    plsc.subcore_barrier()
    _acc_flush(acc_sh, out_hbm, cid, sid, r0)


def kernel(x, edge_index, proj, gin_params):
    Wp, bp = proj
    layers0, layers1 = gin_params
    ei = edge_index.astype(jnp.int32)
    ei0 = ei.reshape(2, NC * NS, NCH0, CB, EB)
    ei1 = ei.reshape(2, NS, NCH1, CB, EB)
    zeros = jnp.zeros((N, DH), jnp.float32)

    h0 = _proj(x, Wp, bp)
    p = _make_segsum0()(h0, zeros, ei0)
    z, *sums = _linear_first0(h0, p, layers0[0][0], layers0[0][1])
    z, *sums = _linear(z, sums, layers0[0][2:], *layers0[1][:2])
    z, *sums = _linear(z, sums, layers0[1][2:], *layers0[2][:2])
    z, *sums = _linear(z, sums, layers0[2][2:], *layers0[3][:2])
    h1a, h1b = _apply(z, sums, layers0[3][2:], split_out=True)

    s1 = _make_segsum1()(h1a, h1b, ei1)
    z, *sums = _linear_first1(s1, layers1[0][0], layers1[0][1])
    z, *sums = _linear(z, sums, layers1[0][2:], *layers1[1][:2])
    z, *sums = _linear(z, sums, layers1[1][2:], *layers1[2][:2])
    z, *sums = _linear(z, sums, layers1[2][2:], *layers1[3][:2])
    return _apply(z, sums, layers1[3][2:], split_out=False)


# speedup vs baseline: 5.7669x; 1.1701x over previous
import functools

import jax
import jax.numpy as jnp
from jax import lax
from jax.experimental import pallas as pl
from jax.experimental.pallas import tpu as pltpu
from jax.experimental.pallas import tpu_sc as plsc

N = 10000
E = 320000
DH = 128
NC = 2
NS = 16

EB1 = 80
NBLK1 = E // (NS * EB1)
EB0 = 50
NBLK0 = E // (NC * NS * EB0)

RPS = 624
RTAIL = N - NS * RPS

BR = 2000
GR = N // BR

_INV_N = 1.0 / N


def _store_sums(i, z, s1_ref, s2_ref):
    ps = jnp.sum(z, axis=0, keepdims=True)
    psq = jnp.sum(z * z, axis=0, keepdims=True)

    @pl.when(i == 0)
    def _():
        s1_ref[...] = ps
        s2_ref[...] = psq

    @pl.when(i != 0)
    def _():
        s1_ref[...] = s1_ref[...] + ps
        s2_ref[...] = s2_ref[...] + psq


def _norm_gelu(z, s1_ref, s2_ref, g_ref, be_ref, al_ref):
    al = al_ref[...]
    m = s1_ref[...] * _INV_N
    e2 = s2_ref[...] * _INV_N
    am = al * m
    var = e2 - 2.0 * am * m + am * am
    zn = (z - am) * jax.lax.rsqrt(var + 1e-5) * g_ref[...] + be_ref[...]
    return jax.nn.gelu(zn)


def _sums_shapes(d):
    return (jax.ShapeDtypeStruct((1, d), jnp.float32),
            jax.ShapeDtypeStruct((1, d), jnp.float32))


def _sums_specs(d):
    return (pl.BlockSpec((1, d), lambda i: (0, 0)),
            pl.BlockSpec((1, d), lambda i: (0, 0)))


def _row_spec(d):
    return pl.BlockSpec((BR, d), lambda i: (i, 0))


def _fs(a):
    return pl.BlockSpec(a.shape, lambda i: (0,) * a.ndim)


def _proj_body(x_ref, wp_ref, bp_ref, o_ref):
    h = jnp.dot(x_ref[...], wp_ref[...], preferred_element_type=jnp.float32)
    o_ref[...] = h + bp_ref[...]


def _proj(x, Wp, bp):
    bp2 = bp.reshape(1, -1)
    return pl.pallas_call(
        _proj_body,
        grid=(GR,),
        in_specs=[_row_spec(Wp.shape[0]), _fs(Wp), _fs(bp2)],
        out_specs=_row_spec(Wp.shape[1]),
        out_shape=jax.ShapeDtypeStruct((N, Wp.shape[1]), jnp.float32),
    )(x, Wp, bp2)


def _first0_body(h_ref, s_ref, w_ref, b_ref, o_ref, s1_ref, s2_ref):
    hin = h_ref[...] + (s_ref[0] + s_ref[1])
    z = jnp.dot(hin, w_ref[...], preferred_element_type=jnp.float32)
    z = z + b_ref[...]
    o_ref[...] = z
    _store_sums(pl.program_id(0), z, s1_ref, s2_ref)


def _linear_first0(h0, s, W, b):
    dout = W.shape[1]
    b2 = b.reshape(1, dout)
    return pl.pallas_call(
        _first0_body,
        grid=(GR,),
        in_specs=[_row_spec(DH),
                  pl.BlockSpec((2, BR, DH), lambda i: (0, i, 0)),
                  _fs(W), _fs(b2)],
        out_specs=(_row_spec(dout),) + _sums_specs(dout),
        out_shape=(jax.ShapeDtypeStruct((N, dout), jnp.float32),)
        + _sums_shapes(dout),
    )(h0, s, W, b2)


def _first1_body(s_ref, w_ref, b_ref, o_ref, s1_ref, s2_ref):
    h = jnp.concatenate([s_ref[0], s_ref[1]], axis=1)
    z = jnp.dot(h, w_ref[...], preferred_element_type=jnp.float32)
    z = z + b_ref[...]
    o_ref[...] = z
    _store_sums(pl.program_id(0), z, s1_ref, s2_ref)


def _linear_first1(s, W, b):
    dout = W.shape[1]
    b2 = b.reshape(1, dout)
    return pl.pallas_call(
        _first1_body,
        grid=(GR,),
        in_specs=[pl.BlockSpec((2, BR, DH), lambda i: (0, i, 0)),
                  _fs(W), _fs(b2)],
        out_specs=(_row_spec(dout),) + _sums_specs(dout),
        out_shape=(jax.ShapeDtypeStruct((N, dout), jnp.float32),)
        + _sums_shapes(dout),
    )(s, W, b2)


def _linear_body(s_ref, z1_ref, z2_ref, g_ref, be_ref, al_ref, w_ref, b_ref,
                 o_ref, s1_ref, s2_ref):
    a = _norm_gelu(s_ref[...], z1_ref, z2_ref, g_ref, be_ref, al_ref)
    zn = jnp.dot(a, w_ref[...], preferred_element_type=jnp.float32)
    zn = zn + b_ref[...]
    o_ref[...] = zn
    _store_sums(pl.program_id(0), zn, s1_ref, s2_ref)


def _linear(z, sums, norm_params, W, b):
    g, be, al = norm_params
    din, dout = W.shape
    onerow = pl.BlockSpec((1, din), lambda i: (0, 0))
    b2 = b.reshape(1, dout)
    return pl.pallas_call(
        _linear_body,
        grid=(GR,),
        in_specs=[_row_spec(din), onerow, onerow, onerow, onerow, onerow,
                  _fs(W), _fs(b2)],
        out_specs=(_row_spec(dout),) + _sums_specs(dout),
        out_shape=(jax.ShapeDtypeStruct((N, dout), jnp.float32),)
        + _sums_shapes(dout),
    )(z, sums[0], sums[1], g.reshape(1, din), be.reshape(1, din),
      al.reshape(1, din), W, b2)


def _apply_body(z_ref, z1_ref, z2_ref, g_ref, be_ref, al_ref, *out_refs):
    a = _norm_gelu(z_ref[...], z1_ref, z2_ref, g_ref, be_ref, al_ref)
    if len(out_refs) == 2:
        d = a.shape[1] // 2
        out_refs[0][...] = a[:, :d]
        out_refs[1][...] = a[:, d:]
    else:
        out_refs[0][...] = a


def _apply(z, sums, norm_params, split_out):
    g, be, al = norm_params
    d = z.shape[1]
    if split_out:
        out_specs = (_row_spec(d // 2), _row_spec(d // 2))
        out_shape = (jax.ShapeDtypeStruct((N, d // 2), jnp.float32),
                     jax.ShapeDtypeStruct((N, d // 2), jnp.float32))
    else:
        out_specs = _row_spec(d)
        out_shape = jax.ShapeDtypeStruct((N, d), jnp.float32)
    onerow = pl.BlockSpec((1, d), lambda i: (0, 0))
    return pl.pallas_call(
        _apply_body,
        grid=(GR,),
        in_specs=[_row_spec(d), onerow, onerow, onerow, onerow, onerow],
        out_specs=out_specs,
        out_shape=out_shape,
    )(z, sums[0], sums[1], g.reshape(1, d), be.reshape(1, d),
      al.reshape(1, d))


def _acc_init(table, acc_sh, sid, r0):
    pltpu.sync_copy(table.at[pl.ds(r0, RPS)], acc_sh.at[pl.ds(r0, RPS)])

    @pl.when(sid == NS - 1)
    def _():
        pltpu.sync_copy(table.at[pl.ds(NS * RPS, RTAIL)],
                        acc_sh.at[pl.ds(NS * RPS, RTAIL)])


def _acc_flush(acc_sh, out_hbm, cid, sid, r0):
    pltpu.sync_copy(acc_sh.at[pl.ds(r0, RPS)], out_hbm.at[cid, pl.ds(r0, RPS)])

    @pl.when(sid == NS - 1)
    def _():
        pltpu.sync_copy(acc_sh.at[pl.ds(NS * RPS, RTAIL)],
                        out_hbm.at[cid, pl.ds(NS * RPS, RTAIL)])


def _edge_pipeline(table, ei_hbm, widx, nblk, acc_sh, bufs):
    (src0, dst0, src1, dst1, rows0, rows1,
     semi0, semi1, semg0, semg1) = bufs

    def idx_fetch(s, d, b, sem):
        pltpu.async_copy(ei_hbm.at[0, widx, b], s, sem)
        pltpu.async_copy(ei_hbm.at[1, widx, b], d, sem)

    def idx_wait(s, d, sem):
        pltpu.make_async_copy(ei_hbm.at[0, widx, 0], s, sem).wait()
        pltpu.make_async_copy(ei_hbm.at[1, widx, 0], d, sem).wait()

    def gather_start(s, rows, sem):
        pltpu.async_copy(table.at[s.at[0]], rows, sem)

    def gather_wait(s, rows, sem):
        pltpu.make_async_copy(table.at[s.at[0]], rows, sem).wait()

    idx_fetch(src0, dst0, 0, semi0)
    idx_fetch(src1, dst1, 1, semi1)
    idx_wait(src0, dst0, semi0)
    gather_start(src0, rows0, semg0)

    @pl.loop(0, nblk // 2)
    def _(jp):
        b0 = 2 * jp
        nb0 = lax.rem(b0 + 2, nblk)
        nb1 = lax.rem(b0 + 3, nblk)
        gather_wait(src0, rows0, semg0)
        idx_wait(src1, dst1, semi1)
        gather_start(src1, rows1, semg1)
        pltpu.sync_copy(rows0, acc_sh.at[dst0.at[0]], add=True)
        idx_fetch(src0, dst0, nb0, semi0)
        gather_wait(src1, rows1, semg1)
        idx_wait(src0, dst0, semi0)
        gather_start(src0, rows0, semg0)
        pltpu.sync_copy(rows1, acc_sh.at[dst1.at[0]], add=True)
        idx_fetch(src1, dst1, nb1, semi1)

    idx_wait(src1, dst1, semi1)
    gather_wait(src0, rows0, semg0)


@functools.cache
def _make_segsum0():
    mesh = plsc.VectorSubcoreMesh(core_axis_name="c", subcore_axis_name="s")
    return jax.jit(functools.partial(
        pl.kernel,
        out_type=jax.ShapeDtypeStruct((NC, N, DH), jnp.float32),
        mesh=mesh,
        scratch_types=[
            pltpu.VMEM((1, EB0), jnp.int32),
            pltpu.VMEM((1, EB0), jnp.int32),
            pltpu.VMEM((1, EB0), jnp.int32),
            pltpu.VMEM((1, EB0), jnp.int32),
            pltpu.VMEM((EB0, DH), jnp.float32),
            pltpu.VMEM((EB0, DH), jnp.float32),
            pltpu.VMEM_SHARED((N, DH), jnp.float32),
            pltpu.SemaphoreType.DMA,
            pltpu.SemaphoreType.DMA,
            pltpu.SemaphoreType.DMA,
            pltpu.SemaphoreType.DMA,
        ],
    )(_segsum0_body))


def _segsum0_body(h_hbm, zeros_hbm, ei_hbm, out_hbm, *bufs):
    acc_sh = bufs[6]
    cid = lax.axis_index("c")
    sid = lax.axis_index("s")
    w = cid * NS + sid
    r0 = sid * RPS
    _acc_init(zeros_hbm, acc_sh, sid, r0)
    plsc.subcore_barrier()
    _edge_pipeline(h_hbm, ei_hbm, w, NBLK0, acc_sh,
                   bufs[:6] + bufs[7:])
    plsc.subcore_barrier()
    _acc_flush(acc_sh, out_hbm, cid, sid, r0)


@functools.cache
def _make_segsum1():
    mesh = plsc.VectorSubcoreMesh(core_axis_name="c", subcore_axis_name="s")
    return jax.jit(functools.partial(
        pl.kernel,
        out_type=jax.ShapeDtypeStruct((NC, N, DH), jnp.float32),
        mesh=mesh,
        scratch_types=[
            pltpu.VMEM((1, EB1), jnp.int32),
            pltpu.VMEM((1, EB1), jnp.int32),
            pltpu.VMEM((1, EB1), jnp.int32),
            pltpu.VMEM((1, EB1), jnp.int32),
            pltpu.VMEM((EB1, DH), jnp.float32),
            pltpu.VMEM((EB1, DH), jnp.float32),
            pltpu.VMEM_SHARED((N, DH), jnp.float32),
            pltpu.SemaphoreType.DMA,
            pltpu.SemaphoreType.DMA,
            pltpu.SemaphoreType.DMA,
            pltpu.SemaphoreType.DMA,
        ],
    )(_segsum1_body))


def _segsum1_body(ha_hbm, hb_hbm, ei_hbm, out_hbm, *bufs):
    acc_sh = bufs[6]
    cid = lax.axis_index("c")
    sid = lax.axis_index("s")
    r0 = sid * RPS

    def run(table):
        _acc_init(table, acc_sh, sid, r0)
        plsc.subcore_barrier()
        _edge_pipeline(table, ei_hbm, sid, NBLK1, acc_sh,
                       bufs[:6] + bufs[7:])

    @pl.when(cid == 0)
    def _():
        run(ha_hbm)

    @pl.when(cid == 1)
    def _():
        run(hb_hbm)

    plsc.subcore_barrier()
    _acc_flush(acc_sh, out_hbm, cid, sid, r0)


def kernel(x, edge_index, proj, gin_params):
    Wp, bp = proj
    layers0, layers1 = gin_params
    ei = edge_index.astype(jnp.int32)
    ei0 = ei.reshape(2, NC * NS, NBLK0, 1, EB0)
    ei1 = ei.reshape(2, NS, NBLK1, 1, EB1)
    zeros = jnp.zeros((N, DH), jnp.float32)

    h0 = _proj(x, Wp, bp)
    p = _make_segsum0()(h0, zeros, ei0)
    z, *sums = _linear_first0(h0, p, layers0[0][0], layers0[0][1])
    z, *sums = _linear(z, sums, layers0[0][2:], *layers0[1][:2])
    z, *sums = _linear(z, sums, layers0[1][2:], *layers0[2][:2])
    z, *sums = _linear(z, sums, layers0[2][2:], *layers0[3][:2])
    h1a, h1b = _apply(z, sums, layers0[3][2:], split_out=True)

    s1 = _make_segsum1()(h1a, h1b, ei1)
    z, *sums = _linear_first1(s1, layers1[0][0], layers1[0][1])
    z, *sums = _linear(z, sums, layers1[0][2:], *layers1[1][:2])
    z, *sums = _linear(z, sums, layers1[1][2:], *layers1[2][:2])
    z, *sums = _linear(z, sums, layers1[2][2:], *layers1[3][:2])
    return _apply(z, sums, layers1[3][2:], split_out=False)


# speedup vs baseline: 7.6898x; 1.3334x over previous
import functools

import jax
import jax.numpy as jnp
from jax import lax
from jax.experimental import pallas as pl
from jax.experimental.pallas import tpu as pltpu
from jax.experimental.pallas import tpu_sc as plsc

N = 10000
E = 320000
DH = 128
NC = 2
NS = 16

EB1 = 80
NBLK1 = E // (NS * EB1)
EB0 = 80
NBLK0 = E // (NC * NS * EB0)

RPS = 624
RTAIL = N - NS * RPS

BR = 2000
GR = N // BR

_INV_N = 1.0 / N


def _store_sums(i, z, s1_ref, s2_ref):
    ps = jnp.sum(z, axis=0, keepdims=True)
    psq = jnp.sum(z * z, axis=0, keepdims=True)

    @pl.when(i == 0)
    def _():
        s1_ref[...] = ps
        s2_ref[...] = psq

    @pl.when(i != 0)
    def _():
        s1_ref[...] = s1_ref[...] + ps
        s2_ref[...] = s2_ref[...] + psq


def _norm_gelu(z, s1_ref, s2_ref, g_ref, be_ref, al_ref):
    al = al_ref[...]
    m = s1_ref[...] * _INV_N
    e2 = s2_ref[...] * _INV_N
    am = al * m
    var = e2 - 2.0 * am * m + am * am
    zn = (z - am) * jax.lax.rsqrt(var + 1e-5) * g_ref[...] + be_ref[...]
    return jax.nn.gelu(zn)


def _sums_shapes(d):
    return (jax.ShapeDtypeStruct((1, d), jnp.float32),
            jax.ShapeDtypeStruct((1, d), jnp.float32))


def _sums_specs(d):
    return (pl.BlockSpec((1, d), lambda i: (0, 0)),
            pl.BlockSpec((1, d), lambda i: (0, 0)))


def _row_spec(d):
    return pl.BlockSpec((BR, d), lambda i: (i, 0))


def _fs(a):
    return pl.BlockSpec(a.shape, lambda i: (0,) * a.ndim)


def _proj_body(x_ref, wp_ref, bp_ref, o_ref):
    h = jnp.dot(x_ref[...], wp_ref[...], preferred_element_type=jnp.float32)
    o_ref[...] = h + bp_ref[...]


def _proj(x, Wp, bp):
    bp2 = bp.reshape(1, -1)
    return pl.pallas_call(
        _proj_body,
        grid=(GR,),
        in_specs=[_row_spec(Wp.shape[0]), _fs(Wp), _fs(bp2)],
        out_specs=_row_spec(Wp.shape[1]),
        out_shape=jax.ShapeDtypeStruct((N, Wp.shape[1]), jnp.float32),
    )(x, Wp, bp2)


def _first0_body(h_ref, s_ref, w_ref, b_ref, o_ref, s1_ref, s2_ref):
    hin = h_ref[...] + (s_ref[0] + s_ref[1])
    z = jnp.dot(hin, w_ref[...], preferred_element_type=jnp.float32)
    z = z + b_ref[...]
    o_ref[...] = z
    _store_sums(pl.program_id(0), z, s1_ref, s2_ref)


def _linear_first0(h0, s, W, b):
    dout = W.shape[1]
    b2 = b.reshape(1, dout)
    return pl.pallas_call(
        _first0_body,
        grid=(GR,),
        in_specs=[_row_spec(DH),
                  pl.BlockSpec((2, BR, DH), lambda i: (0, i, 0)),
                  _fs(W), _fs(b2)],
        out_specs=(_row_spec(dout),) + _sums_specs(dout),
        out_shape=(jax.ShapeDtypeStruct((N, dout), jnp.float32),)
        + _sums_shapes(dout),
    )(h0, s, W, b2)


def _first1_body(s_ref, w_ref, b_ref, o_ref, s1_ref, s2_ref):
    h = jnp.concatenate([s_ref[0], s_ref[1]], axis=1)
    z = jnp.dot(h, w_ref[...], preferred_element_type=jnp.float32)
    z = z + b_ref[...]
    o_ref[...] = z
    _store_sums(pl.program_id(0), z, s1_ref, s2_ref)


def _linear_first1(s, W, b):
    dout = W.shape[1]
    b2 = b.reshape(1, dout)
    return pl.pallas_call(
        _first1_body,
        grid=(GR,),
        in_specs=[pl.BlockSpec((2, BR, DH), lambda i: (0, i, 0)),
                  _fs(W), _fs(b2)],
        out_specs=(_row_spec(dout),) + _sums_specs(dout),
        out_shape=(jax.ShapeDtypeStruct((N, dout), jnp.float32),)
        + _sums_shapes(dout),
    )(s, W, b2)


def _linear_body(s_ref, z1_ref, z2_ref, g_ref, be_ref, al_ref, w_ref, b_ref,
                 o_ref, s1_ref, s2_ref):
    a = _norm_gelu(s_ref[...], z1_ref, z2_ref, g_ref, be_ref, al_ref)
    zn = jnp.dot(a, w_ref[...], preferred_element_type=jnp.float32)
    zn = zn + b_ref[...]
    o_ref[...] = zn
    _store_sums(pl.program_id(0), zn, s1_ref, s2_ref)


def _linear(z, sums, norm_params, W, b):
    g, be, al = norm_params
    din, dout = W.shape
    onerow = pl.BlockSpec((1, din), lambda i: (0, 0))
    b2 = b.reshape(1, dout)
    return pl.pallas_call(
        _linear_body,
        grid=(GR,),
        in_specs=[_row_spec(din), onerow, onerow, onerow, onerow, onerow,
                  _fs(W), _fs(b2)],
        out_specs=(_row_spec(dout),) + _sums_specs(dout),
        out_shape=(jax.ShapeDtypeStruct((N, dout), jnp.float32),)
        + _sums_shapes(dout),
    )(z, sums[0], sums[1], g.reshape(1, din), be.reshape(1, din),
      al.reshape(1, din), W, b2)


def _apply_body(z_ref, z1_ref, z2_ref, g_ref, be_ref, al_ref, *out_refs):
    a = _norm_gelu(z_ref[...], z1_ref, z2_ref, g_ref, be_ref, al_ref)
    if len(out_refs) == 2:
        d = a.shape[1] // 2
        out_refs[0][...] = a[:, :d]
        out_refs[1][...] = a[:, d:]
    else:
        out_refs[0][...] = a


def _apply(z, sums, norm_params, split_out):
    g, be, al = norm_params
    d = z.shape[1]
    if split_out:
        out_specs = (_row_spec(d // 2), _row_spec(d // 2))
        out_shape = (jax.ShapeDtypeStruct((N, d // 2), jnp.float32),
                     jax.ShapeDtypeStruct((N, d // 2), jnp.float32))
    else:
        out_specs = _row_spec(d)
        out_shape = jax.ShapeDtypeStruct((N, d), jnp.float32)
    onerow = pl.BlockSpec((1, d), lambda i: (0, 0))
    return pl.pallas_call(
        _apply_body,
        grid=(GR,),
        in_specs=[_row_spec(d), onerow, onerow, onerow, onerow, onerow],
        out_specs=out_specs,
        out_shape=out_shape,
    )(z, sums[0], sums[1], g.reshape(1, d), be.reshape(1, d),
      al.reshape(1, d))


def _acc_init(table, acc_sh, sid, r0):
    pltpu.sync_copy(table.at[pl.ds(r0, RPS)], acc_sh.at[pl.ds(r0, RPS)])

    @pl.when(sid == NS - 1)
    def _():
        pltpu.sync_copy(table.at[pl.ds(NS * RPS, RTAIL)],
                        acc_sh.at[pl.ds(NS * RPS, RTAIL)])


def _acc_flush(acc_sh, out_hbm, cid, sid, r0):
    pltpu.sync_copy(acc_sh.at[pl.ds(r0, RPS)], out_hbm.at[cid, pl.ds(r0, RPS)])

    @pl.when(sid == NS - 1)
    def _():
        pltpu.sync_copy(acc_sh.at[pl.ds(NS * RPS, RTAIL)],
                        out_hbm.at[cid, pl.ds(NS * RPS, RTAIL)])


def _edge_pipeline(table, ei_hbm, widx, nblk, acc_sh, bufs):
    (src, dst, rows, semi, semg) = bufs

    def idx_fetch(k, b):
        pltpu.async_copy(ei_hbm.at[0, widx, b], src[k], semi[k])
        pltpu.async_copy(ei_hbm.at[1, widx, b], dst[k], semi[k])

    def idx_wait(k):
        pltpu.make_async_copy(ei_hbm.at[0, widx, 0], src[k], semi[k]).wait()
        pltpu.make_async_copy(ei_hbm.at[1, widx, 0], dst[k], semi[k]).wait()

    def gather_start(k, r):
        pltpu.async_copy(table.at[src[k].at[0]], rows[r], semg[r])

    def gather_wait(r):
        pltpu.make_async_copy(table.at[src[0].at[0]], rows[r],
                              semg[r]).wait()

    def scatter(r, k):
        pltpu.sync_copy(rows[r], acc_sh.at[dst[k].at[0]], add=True)

    ngrp = nblk // 4
    tail = nblk - 4 * ngrp

    for k in range(4):
        idx_fetch(k, k)
    idx_wait(0)
    gather_start(0, 0)
    idx_wait(1)
    gather_start(1, 1)

    @pl.loop(0, ngrp)
    def _(g):
        b = 4 * g
        for k in range(4):
            r = k % 2
            gather_wait(r)
            scatter(r, k)
            idx_fetch(k, lax.rem(b + k + 4, nblk))
            idx_wait((k + 2) % 4)
            gather_start((k + 2) % 4, r)

    for k in range(tail):
        gather_wait(k % 2)
        scatter(k % 2, k)

    idx_wait(2)
    idx_wait(3)
    if tail == 0:
        gather_wait(0)
        gather_wait(1)
    elif tail == 1:
        gather_wait(1)


@functools.cache
def _make_segsum0():
    mesh = plsc.VectorSubcoreMesh(core_axis_name="c", subcore_axis_name="s")
    return jax.jit(functools.partial(
        pl.kernel,
        out_type=jax.ShapeDtypeStruct((NC, N, DH), jnp.float32),
        mesh=mesh,
        scratch_types=_sc_scratch(EB0),
    )(_segsum0_body))


def _sc_scratch(eb):
    return ([pltpu.VMEM((1, eb), jnp.int32) for _ in range(8)]
            + [pltpu.VMEM((eb, DH), jnp.float32) for _ in range(2)]
            + [pltpu.VMEM_SHARED((N, DH), jnp.float32)]
            + [pltpu.SemaphoreType.DMA for _ in range(6)])


def _sc_bufs(bufs):
    src = bufs[0:8:2]
    dst = bufs[1:8:2]
    rows = bufs[8:10]
    acc_sh = bufs[10]
    semi = bufs[11:15]
    semg = bufs[15:17]
    return acc_sh, (src, dst, rows, semi, semg)


def _segsum0_body(h_hbm, zeros_hbm, ei_hbm, out_hbm, *bufs):
    acc_sh, pipe_bufs = _sc_bufs(bufs)
    cid = lax.axis_index("c")
    sid = lax.axis_index("s")
    w = cid * NS + sid
    r0 = sid * RPS
    _acc_init(zeros_hbm, acc_sh, sid, r0)
    plsc.subcore_barrier()
    _edge_pipeline(h_hbm, ei_hbm, w, NBLK0, acc_sh, pipe_bufs)
    plsc.subcore_barrier()
    _acc_flush(acc_sh, out_hbm, cid, sid, r0)


@functools.cache
def _make_segsum1():
    mesh = plsc.VectorSubcoreMesh(core_axis_name="c", subcore_axis_name="s")
    return jax.jit(functools.partial(
        pl.kernel,
        out_type=jax.ShapeDtypeStruct((NC, N, DH), jnp.float32),
        mesh=mesh,
        scratch_types=_sc_scratch(EB1),
    )(_segsum1_body))


def _segsum1_body(ha_hbm, hb_hbm, ei_hbm, out_hbm, *bufs):
    acc_sh, pipe_bufs = _sc_bufs(bufs)
    cid = lax.axis_index("c")
    sid = lax.axis_index("s")
    r0 = sid * RPS

    def run(table):
        _acc_init(table, acc_sh, sid, r0)
        plsc.subcore_barrier()
        _edge_pipeline(table, ei_hbm, sid, NBLK1, acc_sh, pipe_bufs)

    @pl.when(cid == 0)
    def _():
        run(ha_hbm)

    @pl.when(cid == 1)
    def _():
        run(hb_hbm)

    plsc.subcore_barrier()
    _acc_flush(acc_sh, out_hbm, cid, sid, r0)


def kernel(x, edge_index, proj, gin_params):
    Wp, bp = proj
    layers0, layers1 = gin_params
    ei = edge_index.astype(jnp.int32)
    ei0 = ei.reshape(2, NC * NS, NBLK0, 1, EB0)
    ei1 = ei.reshape(2, NS, NBLK1, 1, EB1)
    zeros = jnp.zeros((N, DH), jnp.float32)

    h0 = _proj(x, Wp, bp)
    p = _make_segsum0()(h0, zeros, ei0)
    z, *sums = _linear_first0(h0, p, layers0[0][0], layers0[0][1])
    z, *sums = _linear(z, sums, layers0[0][2:], *layers0[1][:2])
    z, *sums = _linear(z, sums, layers0[1][2:], *layers0[2][:2])
    z, *sums = _linear(z, sums, layers0[2][2:], *layers0[3][:2])
    h1a, h1b = _apply(z, sums, layers0[3][2:], split_out=True)

    s1 = _make_segsum1()(h1a, h1b, ei1)
    z, *sums = _linear_first1(s1, layers1[0][0], layers1[0][1])
    z, *sums = _linear(z, sums, layers1[0][2:], *layers1[1][:2])
    z, *sums = _linear(z, sums, layers1[1][2:], *layers1[2][:2])
    z, *sums = _linear(z, sums, layers1[2][2:], *layers1[3][:2])
    return _apply(z, sums, layers1[3][2:], split_out=False)
